# baseline (speedup 1.0000x reference)
import jax
import jax.numpy as jnp
from jax import lax
from jax.experimental import pallas as pl
from jax.experimental.pallas import tpu as pltpu

MIN_X = -10.0
MAX_X = 70.0
MIN_Y = -40.0
MAX_Y = 40.0
PPM = 4
NX = 320
NY = 320
NPIL = NY * NX


def _make_body(npts, ch, nch, f1, f2):
    def body(np_ref, lidar_ref, w1_ref, b1_ref, w2_ref, b2_ref,
             out_ref, s_ref, pidx_ref, xyz1_ref, mg_ref, featc_ref, sem):
        b = pl.program_id(0)
        p = pl.program_id(1)
        c = pl.program_id(2)
        n = jnp.minimum(jnp.maximum(np_ref[b], 0), npts)
        base = c * ch
        hi = jnp.clip(n - base, 0, ch)

        @pl.when((p == 0) & (c == 0))
        def _():
            s_ref[...] = jnp.zeros_like(s_ref)

        pts = lidar_ref[...]
        xcol = pts[:, 0:1]
        ycol = pts[:, 1:2]
        pid = base + lax.broadcasted_iota(jnp.int32, (ch, 1), 0)
        inb = ((xcol >= MIN_X) & (xcol < MAX_X)
               & (ycol >= MIN_Y) & (ycol < MAX_Y))
        keep = (pid < n) & inb
        gx = jnp.clip(((xcol - MIN_X) * PPM).astype(jnp.int32), 0, NX - 1)
        gy = jnp.clip(((ycol - MIN_Y) * PPM).astype(jnp.int32), 0, NY - 1)
        pidx_ref[...] = (NY - 1 - gx) * NX + gy
        m = keep.astype(jnp.float32)

        @pl.when(p == 0)
        def _():
            xyz1_ref[:, 0:3] = pts[:, 0:3] * m
            xyz1_ref[:, 3:4] = m

            def add_body(i, cc):
                q = pidx_ref[i, 0]
                s_ref[pl.ds(q, 1), 32:36] += xyz1_ref[pl.ds(i, 1), :]
                return cc

            lax.fori_loop(0, hi, add_body, 0)

        @pl.when(p == 1)
        def _():
            def g_body(i, cc):
                q = pidx_ref[i, 0]
                mg_ref[pl.ds(i, 1), :] = s_ref[pl.ds(q, 1), 32:36]
                return cc

            lax.fori_loop(0, hi, g_body, 0)

            mgc = mg_ref[...]
            cnt = jnp.maximum(mgc[:, 3:4], 1.0)
            clus = pts[:, 0:3] - mgc[:, 0:3] / cnt
            xp = xcol - (gy.astype(jnp.float32) / PPM + MIN_X)
            yp = ycol - (gx.astype(jnp.float32) / PPM + MIN_Y)
            feats = jnp.concatenate([pts, clus, xp, yp], axis=1)
            h = jnp.dot(feats, w1_ref[...],
                        preferred_element_type=jnp.float32) + b1_ref[...]
            h = jnp.maximum(h, 0.0)
            h = jnp.dot(h, w2_ref[...],
                        preferred_element_type=jnp.float32) + b2_ref[...]
            h = jnp.maximum(h, 0.0)
            featc_ref[...] = jnp.where(keep, h, 0.0)

            def s_body(i, cc):
                q = pidx_ref[i, 0]
                s_ref[pl.ds(q, 1), 0:32] = jnp.maximum(
                    s_ref[pl.ds(q, 1), 0:32], featc_ref[pl.ds(i, 1), :])
                return cc

            lax.fori_loop(0, hi, s_body, 0)

            @pl.when(c == nch - 1)
            def _():
                cp = pltpu.make_async_copy(s_ref, out_ref.at[b], sem)
                cp.start()
                cp.wait()

    return body


def kernel(lidar_list, num_points, W1, b1, gamma1, beta1, rm1, rv1,
           W2, b2, gamma2, beta2, rm2, rv2):
    eps = 1e-5
    f1 = W1.shape[0]
    f2 = W2.shape[0]
    s1 = gamma1 / jnp.sqrt(rv1 + eps)
    w1t = (W1 * s1[:, None]).T
    b1f = ((b1 - rm1) * s1 + beta1).reshape(1, f1)
    s2 = gamma2 / jnp.sqrt(rv2 + eps)
    w2t = (W2 * s2[:, None]).T
    b2f = ((b2 - rm2) * s2 + beta2).reshape(1, f2)
    np32 = num_points.astype(jnp.int32)

    nb, npts, cin = lidar_list.shape
    ch = 1000 if npts % 1000 == 0 else npts
    nch = npts // ch

    out = pl.pallas_call(
        _make_body(npts, ch, nch, f1, f2),
        grid_spec=pltpu.PrefetchScalarGridSpec(
            num_scalar_prefetch=1,
            grid=(nb, 2, nch),
            in_specs=[
                pl.BlockSpec((None, ch, cin), lambda b, p, c, *_: (b, c, 0)),
                pl.BlockSpec((9, f1), lambda b, p, c, *_: (0, 0)),
                pl.BlockSpec((1, f1), lambda b, p, c, *_: (0, 0)),
                pl.BlockSpec((f1, f2), lambda b, p, c, *_: (0, 0)),
                pl.BlockSpec((1, f2), lambda b, p, c, *_: (0, 0)),
            ],
            out_specs=pl.BlockSpec(memory_space=pl.ANY),
            scratch_shapes=[
                pltpu.VMEM((NPIL, 128), jnp.float32),
                pltpu.VMEM((ch, 1), jnp.int32),
                pltpu.VMEM((ch, 4), jnp.float32),
                pltpu.VMEM((ch, 4), jnp.float32),
                pltpu.VMEM((ch, f2), jnp.float32),
                pltpu.SemaphoreType.DMA,
            ],
        ),
        out_shape=jax.ShapeDtypeStruct((nb, NPIL, 128), jnp.float32),
    )(np32, lidar_list, w1t, b1f, w2t, b2f)
    canvas = out[:, :, :f2].transpose(0, 2, 1)
    return canvas.reshape(nb, f2, NY, NX)
